# Initial kernel scaffold; baseline (speedup 1.0000x reference)
import jax
import jax.numpy as jnp
from jax.experimental import pallas as pl


def kernel(x, table):
    raise NotImplementedError("write your pallas kernel here")


# speedup vs baseline: 11.3698x; 11.3698x over previous
import jax
import jax.numpy as jnp
from jax import lax
from jax.experimental import pallas as pl
from jax.experimental.pallas import tpu as pltpu
from jax.experimental.pallas import tpu_sc as plsc

NC, NS, L = 2, 16, 16
NW = NC * NS
B, SEQ, T, D = 1024, 50, 26, 32
N = B * SEQ
PW = N // NW
G = 16
NG = PW // G
NBUF = 4
BLK = T * D
GRP = G * BLK


def _body(x_hbm, tab_hbm, out_hbm, x_v, t2_v, stg_v, s0, s1, s2, s3):
    sems = (s0, s1, s2, s3)
    wid = lax.axis_index("s") * NC + lax.axis_index("c")
    base = wid * PW

    pltpu.sync_copy(x_hbm.at[pl.ds(base, PW)], x_v)
    pltpu.sync_copy(tab_hbm.at[pl.ds(0, 2 * D)], t2_v)

    halves = [t2_v[pl.ds(h * L, L)] for h in range(4)]

    def init_row(i, _):
        stg_v[pl.ds(i * D, L)] = halves[0]
        stg_v[pl.ds(i * D + L, L)] = halves[1]
        return _
    lax.fori_loop(0, NBUF * G * T, init_row, None)

    def stamp(g, b, r):
        va, vb = halves[2 * r], halves[2 * r + 1]
        xv = x_v[pl.ds(g * G, G)]
        for l in range(G):
            off = b * GRP + (l * T + xv[l]) * D
            stg_v[pl.ds(off, L)] = va
            stg_v[pl.ds(off + L, L)] = vb

    def send(g, b):
        return pltpu.async_copy(
            stg_v.at[pl.ds(b * GRP, GRP)],
            out_hbm.at[pl.ds((base + g * G) * BLK, GRP)],
            sems[b])

    for b in range(NBUF):
        stamp(b, b, 1)
        send(b, b)

    def outer(o, _):
        for b in range(NBUF):
            g = NBUF + o * NBUF + b
            pltpu.make_async_copy(
                stg_v.at[pl.ds(b * GRP, GRP)],
                out_hbm.at[pl.ds((base + (g - NBUF) * G) * BLK, GRP)],
                sems[b]).wait()
            stamp(g - NBUF, b, 0)
            stamp(g, b, 1)
            send(g, b)
        return _
    lax.fori_loop(0, (NG - NBUF) // NBUF, outer, None)

    for b in range(NBUF):
        g = NG - NBUF + b
        pltpu.make_async_copy(
            stg_v.at[pl.ds(b * GRP, GRP)],
            out_hbm.at[pl.ds((base + g * G) * BLK, GRP)],
            sems[b]).wait()


@jax.jit
def _run(x_flat, tab_flat):
    mesh = plsc.VectorSubcoreMesh(
        core_axis_name="c", subcore_axis_name="s",
        num_cores=NC, num_subcores=NS)
    return pl.kernel(
        _body,
        out_type=jax.ShapeDtypeStruct((N * BLK,), jnp.float32),
        mesh=mesh,
        scratch_types=[
            pltpu.VMEM((PW,), jnp.int32),
            pltpu.VMEM((2 * D,), jnp.float32),
            pltpu.VMEM((NBUF * GRP,), jnp.float32),
            pltpu.SemaphoreType.DMA,
            pltpu.SemaphoreType.DMA,
            pltpu.SemaphoreType.DMA,
            pltpu.SemaphoreType.DMA,
        ],
    )(x_flat, tab_flat)


def kernel(x, table):
    out = _run(x.reshape(-1), table.reshape(-1))
    return out.reshape(B, SEQ, T, D)


# speedup vs baseline: 160.1134x; 14.0823x over previous
import jax
import jax.numpy as jnp
from jax import lax
from jax.experimental import pallas as pl
from jax.experimental.pallas import tpu as pltpu

B, SEQ, T, D = 1024, 50, 26, 32


def _tc_body(xt_ref, t0_ref, t1_ref, o_ref):
    xs = xt_ref[0, 0, :]
    tt = lax.broadcasted_iota(jnp.int32, (T, B), 0)
    mask = xs[None, :] == tt
    t0 = t0_ref[...]
    t1 = t1_ref[...]
    o_ref[0] = jnp.where(mask[:, None, :], t1[None], t0[None])


@jax.jit
def _run_tc(xt, t0b, t1b):
    return pl.pallas_call(
        _tc_body,
        out_shape=jax.ShapeDtypeStruct((SEQ, T, D, B), jnp.float32),
        grid=(SEQ,),
        in_specs=[
            pl.BlockSpec((1, 1, B), lambda s: (s, 0, 0)),
            pl.BlockSpec((D, B), lambda s: (0, 0)),
            pl.BlockSpec((D, B), lambda s: (0, 0)),
        ],
        out_specs=pl.BlockSpec((1, T, D, B), lambda s: (s, 0, 0, 0)),
    )(xt, t0b, t1b)


def kernel(x, table):
    xt = x.T.reshape(SEQ, 1, B)
    t0b = jnp.broadcast_to(table[0][:, None], (D, B))
    t1b = jnp.broadcast_to(table[1][:, None], (D, B))
    o = _run_tc(xt, t0b, t1b)
    return o.transpose(3, 0, 1, 2)


# speedup vs baseline: 162.9009x; 1.0174x over previous
import jax
import jax.numpy as jnp
from jax import lax
from jax.experimental import pallas as pl
from jax.experimental.pallas import tpu as pltpu

B, SEQ, T, D = 1024, 50, 26, 32
SB = 2


def _tc_body(xt_ref, t0_ref, t1_ref, o_ref):
    tt = lax.broadcasted_iota(jnp.int32, (T, B), 0)
    t0 = t0_ref[...]
    t1 = t1_ref[...]
    for j in range(SB):
        mask = xt_ref[j, 0, :][None, :] == tt
        o_ref[j] = jnp.where(mask[:, None, :], t1[None], t0[None])


@jax.jit
def _run_tc(xt, t0b, t1b):
    return pl.pallas_call(
        _tc_body,
        out_shape=jax.ShapeDtypeStruct((SEQ, T, D, B), jnp.float32),
        grid=(SEQ // SB,),
        in_specs=[
            pl.BlockSpec((SB, 1, B), lambda s: (s, 0, 0)),
            pl.BlockSpec((D, B), lambda s: (0, 0)),
            pl.BlockSpec((D, B), lambda s: (0, 0)),
        ],
        out_specs=pl.BlockSpec((SB, T, D, B), lambda s: (s, 0, 0, 0)),
    )(xt, t0b, t1b)


def kernel(x, table):
    xt = x.T.reshape(SEQ, 1, B)
    t0b = jnp.broadcast_to(table[0][:, None], (D, B))
    t1b = jnp.broadcast_to(table[1][:, None], (D, B))
    o = _run_tc(xt, t0b, t1b)
    return o.transpose(3, 0, 1, 2)
